# Initial kernel scaffold; baseline (speedup 1.0000x reference)
import jax
import jax.numpy as jnp
from jax.experimental import pallas as pl


def kernel(feat_src, feat_dst, edge_index, W, b):
    raise NotImplementedError("write your pallas kernel here")


# speedup vs baseline: 7.5103x; 7.5103x over previous
import functools

import jax
import jax.numpy as jnp
from jax import lax
from jax.experimental import pallas as pl
from jax.experimental.pallas import tpu as pltpu
from jax.experimental.pallas import tpu_sc as plsc

N = 10000
E = 320000
D = 128

NC = 2
NS = 16
NW = NC * NS
CE = E // NW
B = 80
NCH = CE // B
G = B // 16
PS = 10240
SLAB = PS // NS

_mesh = plsc.VectorSubcoreMesh(core_axis_name="c", subcore_axis_name="s")
_sc_params = pltpu.CompilerParams(needs_layout_passes=False)


@functools.partial(
    pl.kernel,
    out_type=[
        jax.ShapeDtypeStruct((E,), jnp.float32),
        jax.ShapeDtypeStruct((NW * 16,), jnp.float32),
    ],
    mesh=_mesh,
    compiler_params=_sc_params,
    scratch_types=[
        pltpu.VMEM((B,), jnp.int32),
        pltpu.VMEM((B,), jnp.int32),
        pltpu.VMEM((B, D), jnp.float32),
        pltpu.VMEM((B, D), jnp.float32),
        pltpu.VMEM((B,), jnp.float32),
        pltpu.VMEM((16,), jnp.float32),
        pltpu.SemaphoreType.DMA,
        pltpu.SemaphoreType.DMA,
    ],
)
def _scores_sc(src_hbm, dst_hbm, fs_hbm, fd_hbm, scores_hbm, tmax_hbm,
               sidx_v, didx_v, srows_v, drows_v, sc_v, tm_v, sem1, sem2):
    cid = lax.axis_index("c")
    sid = lax.axis_index("s")
    wid = sid * NC + cid
    base = wid * CE
    lanes = lax.iota(jnp.int32, 16)

    def chunk(k, rm):
        off = base + k * B
        pltpu.sync_copy(src_hbm.at[pl.ds(off, B)], sidx_v)
        pltpu.sync_copy(dst_hbm.at[pl.ds(off, B)], didx_v)
        cp1 = pltpu.async_copy(fs_hbm.at[sidx_v], srows_v, sem1)
        cp2 = pltpu.async_copy(fd_hbm.at[didx_v], drows_v, sem2)
        cp1.wait()
        cp2.wait()

        def group(g, rm_in):
            vec = jnp.zeros((16,), jnp.float32)
            for ii in range(16):
                i = g * 16 + ii
                acc = srows_v[i, pl.ds(0, 16)] * drows_v[i, pl.ds(0, 16)]
                for t in range(1, D // 16):
                    sl = pl.ds(t * 16, 16)
                    acc = acc + srows_v[i, sl] * drows_v[i, sl]
                sc = lax.reduce_sum(acc, axes=(0,))
                vec = jnp.where(lanes == ii, sc, vec)
            sc_v[pl.ds(g * 16, 16)] = vec
            return jnp.maximum(rm_in, vec)

        rm = lax.fori_loop(0, G, group, rm)
        pltpu.sync_copy(sc_v, scores_hbm.at[pl.ds(off, B)])
        return rm

    rm0 = jnp.full((16,), -jnp.inf, jnp.float32)
    rm = lax.fori_loop(0, NCH, chunk, rm0)
    m = lax.reduce_max(rm, axes=(0,))
    tm_v[...] = jnp.full((16,), m, jnp.float32)
    pltpu.sync_copy(tm_v, tmax_hbm.at[pl.ds(wid * 16, 16)])


@functools.partial(
    pl.kernel,
    out_type=[
        jax.ShapeDtypeStruct((NC * PS, D), jnp.float32),
        jax.ShapeDtypeStruct((NW * PS,), jnp.float32),
    ],
    mesh=_mesh,
    compiler_params=_sc_params,
    scratch_types=[
        pltpu.VMEM((B,), jnp.int32),
        pltpu.VMEM((B,), jnp.int32),
        pltpu.VMEM((B,), jnp.float32),
        pltpu.VMEM((B, D), jnp.float32),
        pltpu.VMEM((PS,), jnp.float32),
        pltpu.VMEM((NW * 16,), jnp.float32),
        pltpu.VMEM_SHARED((PS, D), jnp.float32),
        pltpu.SemaphoreType.DMA,
    ],
)
def _accum_sc(src_hbm, dst_hbm, fs_hbm, scores_hbm, tmax_hbm,
              zu_hbm, zs_hbm, u_hbm, s_hbm,
              sidx_v, didx_v, ex_v, srows_v, s_loc, tm_v, u_sh, sem1):
    cid = lax.axis_index("c")
    sid = lax.axis_index("s")
    wid = sid * NC + cid
    base = wid * CE

    pltpu.sync_copy(tmax_hbm, tm_v)
    m16 = tm_v[pl.ds(0, 16)]
    for j in range(1, NW):
        m16 = jnp.maximum(m16, tm_v[pl.ds(j * 16, 16)])
    m = lax.reduce_max(m16, axes=(0,))
    msp = jnp.full((16,), m, jnp.float32)

    pltpu.sync_copy(zu_hbm.at[pl.ds(sid * SLAB, SLAB)],
                    u_sh.at[pl.ds(sid * SLAB, SLAB)])
    pltpu.sync_copy(zs_hbm, s_loc)
    plsc.subcore_barrier()

    def chunk(k, carry):
        off = base + k * B
        pltpu.sync_copy(src_hbm.at[pl.ds(off, B)], sidx_v)
        pltpu.sync_copy(dst_hbm.at[pl.ds(off, B)], didx_v)
        cp1 = pltpu.async_copy(fs_hbm.at[sidx_v], srows_v, sem1)
        pltpu.sync_copy(scores_hbm.at[pl.ds(off, B)], ex_v)
        for j in range(G):
            sl = pl.ds(j * 16, 16)
            e16 = jnp.exp(ex_v[sl] - msp)
            ex_v[sl] = e16
            plsc.addupdate_scatter(s_loc, [didx_v[sl]], e16)
        cp1.wait()

        def scale(g, c2):
            e_vec = ex_v[pl.ds(g * 16, 16)]
            for ii in range(16):
                i = g * 16 + ii
                e16 = jnp.full((16,), e_vec[ii], jnp.float32)
                for t in range(D // 16):
                    sl = pl.ds(t * 16, 16)
                    srows_v[i, sl] = srows_v[i, sl] * e16
            return c2

        lax.fori_loop(0, G, scale, 0)
        pltpu.sync_copy(srows_v, u_sh.at[didx_v], add=True)
        return carry

    lax.fori_loop(0, NCH, chunk, 0)
    plsc.subcore_barrier()

    pltpu.sync_copy(u_sh.at[pl.ds(sid * SLAB, SLAB)],
                    u_hbm.at[pl.ds(cid * PS + sid * SLAB, SLAB)])
    pltpu.sync_copy(s_loc, s_hbm.at[pl.ds(wid * PS, PS)])


def _finish_tc_body(u_ref, s_ref, w_ref, b_ref, o_ref):
    u = u_ref[0] + u_ref[1]
    s = jnp.sum(s_ref[...], axis=1)
    s = jnp.where(s == 0.0, 1.0, s)
    un = u / s[:, None]
    acc = lax.dot_general(un, w_ref[...],
                          dimension_numbers=(((1,), (1,)), ((), ())),
                          preferred_element_type=jnp.float32)
    o_ref[...] = jnp.maximum(acc + b_ref[...], 0.0)


RB = 1000


def _finish_tc(u_part, s_part, W, b):
    return pl.pallas_call(
        _finish_tc_body,
        grid=(N // RB,),
        in_specs=[
            pl.BlockSpec((2, RB, D), lambda i: (0, i, 0)),
            pl.BlockSpec((RB, NW), lambda i: (i, 0)),
            pl.BlockSpec((D, D), lambda i: (0, 0)),
            pl.BlockSpec((1, D), lambda i: (0, 0)),
        ],
        out_specs=pl.BlockSpec((RB, D), lambda i: (i, 0)),
        out_shape=jax.ShapeDtypeStruct((N, D), jnp.float32),
    )(u_part, s_part, W, b)


def kernel(feat_src, feat_dst, edge_index, W, b):
    src = edge_index[0].astype(jnp.int32)
    dst = edge_index[1].astype(jnp.int32)
    scores, tmax = _scores_sc(src, dst, feat_src, feat_dst)
    zu = jnp.zeros((PS, D), jnp.float32)
    zs = jnp.zeros((PS,), jnp.float32)
    u_part, s_part = _accum_sc(src, dst, feat_src, scores, tmax, zu, zs)
    u_part = u_part.reshape(NC, PS, D)[:, :N]
    s_part = s_part.reshape(NW, PS)[:, :N].T
    return _finish_tc(u_part, s_part, W, b.reshape(1, D))


# speedup vs baseline: 13.5933x; 1.8099x over previous
import functools

import jax
import jax.numpy as jnp
from jax import lax
from jax.experimental import pallas as pl
from jax.experimental.pallas import tpu as pltpu
from jax.experimental.pallas import tpu_sc as plsc

N = 10000
E = 320000
D = 128

NC = 2
NS = 16
NW = NC * NS
CE = E // NW
B = 80
NCH = CE // B
G = B // 16
PS = 10240
SLAB = PS // NS

_mesh = plsc.VectorSubcoreMesh(core_axis_name="c", subcore_axis_name="s")
_sc_params = pltpu.CompilerParams(needs_layout_passes=False)


@functools.partial(
    pl.kernel,
    out_type=[
        jax.ShapeDtypeStruct((E,), jnp.float32),
        jax.ShapeDtypeStruct((NW * 16,), jnp.float32),
    ],
    mesh=_mesh,
    compiler_params=_sc_params,
    scratch_types=[
        pltpu.VMEM((CE,), jnp.int32),
        pltpu.VMEM((CE,), jnp.int32),
        pltpu.VMEM((B, D), jnp.float32),
        pltpu.VMEM((B, D), jnp.float32),
        pltpu.VMEM((B, D), jnp.float32),
        pltpu.VMEM((B, D), jnp.float32),
        pltpu.VMEM((B,), jnp.float32),
        pltpu.VMEM((16,), jnp.float32),
        pltpu.SemaphoreType.DMA,
        pltpu.SemaphoreType.DMA,
        pltpu.SemaphoreType.DMA,
        pltpu.SemaphoreType.DMA,
    ],
)
def _scores_sc(src_hbm, dst_hbm, fs_hbm, fd_hbm, scores_hbm, tmax_hbm,
               sidx_all, didx_all, sra, srb, dra, drb, sc_v, tm_v,
               ssa, ssb, sda, sdb):
    cid = lax.axis_index("c")
    sid = lax.axis_index("s")
    wid = sid * NC + cid
    base = wid * CE
    lanes = lax.iota(jnp.int32, 16)

    pltpu.sync_copy(src_hbm.at[pl.ds(base, CE)], sidx_all)
    pltpu.sync_copy(dst_hbm.at[pl.ds(base, CE)], didx_all)
    bufs = ((sra, dra, ssa, sda), (srb, drb, ssb, sdb))

    def issue(k, bi):
        sr, dr, ss, sd = bufs[bi]
        pltpu.async_copy(fs_hbm.at[sidx_all.at[pl.ds(k * B, B)]], sr, ss)
        pltpu.async_copy(fd_hbm.at[didx_all.at[pl.ds(k * B, B)]], dr, sd)

    def wait(bi):
        sr, dr, ss, sd = bufs[bi]
        pltpu.make_async_copy(fs_hbm.at[pl.ds(0, B)], sr, ss).wait()
        pltpu.make_async_copy(fd_hbm.at[pl.ds(0, B)], dr, sd).wait()

    def compute(k, bi, rm):
        sr, dr, _, _ = bufs[bi]

        def group(g, rm_in):
            vec = jnp.zeros((16,), jnp.float32)
            for ii in range(16):
                i = g * 16 + ii
                acc = sr[i, pl.ds(0, 16)] * dr[i, pl.ds(0, 16)]
                for t in range(1, D // 16):
                    sl = pl.ds(t * 16, 16)
                    acc = acc + sr[i, sl] * dr[i, sl]
                sc = lax.reduce_sum(acc, axes=(0,))
                vec = jnp.where(lanes == ii, sc, vec)
            sc_v[pl.ds(g * 16, 16)] = vec
            return jnp.maximum(rm_in, vec)

        rm = lax.fori_loop(0, G, group, rm)
        pltpu.sync_copy(sc_v, scores_hbm.at[pl.ds(base + k * B, B)])
        return rm

    issue(0, 0)
    issue(1, 1)
    rm0 = jnp.full((16,), -jnp.inf, jnp.float32)

    def body(t, rm):
        k0 = 2 * t
        wait(0)
        rm = compute(k0, 0, rm)
        issue(k0 + 2, 0)
        k1 = 2 * t + 1
        wait(1)
        rm = compute(k1, 1, rm)

        @pl.when(k1 + 2 < NCH)
        def _():
            issue(k1 + 2, 1)

        return rm

    rm = lax.fori_loop(0, (NCH - 1) // 2, body, rm0)
    wait(0)
    rm = compute(NCH - 1, 0, rm)
    m = lax.reduce_max(rm, axes=(0,))
    tm_v[...] = jnp.full((16,), m, jnp.float32)
    pltpu.sync_copy(tm_v, tmax_hbm.at[pl.ds(wid * 16, 16)])


@functools.partial(
    pl.kernel,
    out_type=[
        jax.ShapeDtypeStruct((NC * PS, D), jnp.float32),
        jax.ShapeDtypeStruct((NW * PS,), jnp.float32),
    ],
    mesh=_mesh,
    compiler_params=_sc_params,
    scratch_types=[
        pltpu.VMEM((CE,), jnp.int32),
        pltpu.VMEM((B,), jnp.int32),
        pltpu.VMEM((B,), jnp.int32),
        pltpu.VMEM((B,), jnp.float32),
        pltpu.VMEM((B,), jnp.float32),
        pltpu.VMEM((B, D), jnp.float32),
        pltpu.VMEM((B, D), jnp.float32),
        pltpu.VMEM((PS,), jnp.float32),
        pltpu.VMEM((NW * 16,), jnp.float32),
        pltpu.VMEM_SHARED((PS, D), jnp.float32),
        pltpu.SemaphoreType.DMA,
        pltpu.SemaphoreType.DMA,
        pltpu.SemaphoreType.DMA,
        pltpu.SemaphoreType.DMA,
    ],
)
def _accum_sc(src_hbm, dst_hbm, fs_hbm, scores_hbm, tmax_hbm,
              zu_hbm, zs_hbm, u_hbm, s_hbm,
              sidx_all, didx_a, didx_b, ex_a, ex_b, sra, srb, s_loc, tm_v,
              u_sh, sra_sem, srb_sem, aux_a_sem, aux_b_sem):
    cid = lax.axis_index("c")
    sid = lax.axis_index("s")
    wid = sid * NC + cid
    base = wid * CE

    pltpu.sync_copy(tmax_hbm, tm_v)
    m16 = tm_v[pl.ds(0, 16)]
    for j in range(1, NW):
        m16 = jnp.maximum(m16, tm_v[pl.ds(j * 16, 16)])
    m = lax.reduce_max(m16, axes=(0,))
    msp = jnp.full((16,), m, jnp.float32)

    pltpu.sync_copy(src_hbm.at[pl.ds(base, CE)], sidx_all)
    pltpu.sync_copy(zu_hbm.at[pl.ds(sid * SLAB, SLAB)],
                    u_sh.at[pl.ds(sid * SLAB, SLAB)])
    pltpu.sync_copy(zs_hbm, s_loc)
    plsc.subcore_barrier()

    bufs = ((sra, didx_a, ex_a, sra_sem, aux_a_sem),
            (srb, didx_b, ex_b, srb_sem, aux_b_sem))

    def issue(k, bi):
        sr, didx, ex, rsem, asem = bufs[bi]
        off = base + k * B
        pltpu.async_copy(fs_hbm.at[sidx_all.at[pl.ds(k * B, B)]], sr, rsem)
        pltpu.async_copy(dst_hbm.at[pl.ds(off, B)], didx, asem)
        pltpu.async_copy(scores_hbm.at[pl.ds(off, B)], ex, asem)

    def wait(bi):
        sr, didx, ex, rsem, asem = bufs[bi]
        pltpu.make_async_copy(fs_hbm.at[pl.ds(0, B)], sr, rsem).wait()
        pltpu.make_async_copy(dst_hbm.at[pl.ds(0, B)], didx, asem).wait()
        pltpu.make_async_copy(scores_hbm.at[pl.ds(0, B)], ex, asem).wait()

    def compute(k, bi):
        sr, didx, ex, _, _ = bufs[bi]

        def scale(g, c2):
            sl16 = pl.ds(g * 16, 16)
            e_vec = jnp.exp(ex[sl16] - msp)
            plsc.addupdate_scatter(s_loc, [didx[sl16]], e_vec)
            for ii in range(16):
                i = g * 16 + ii
                e16 = jnp.full((16,), e_vec[ii], jnp.float32)
                for t in range(D // 16):
                    sl = pl.ds(t * 16, 16)
                    sr[i, sl] = sr[i, sl] * e16
            return c2

        lax.fori_loop(0, G, scale, 0)
        pltpu.sync_copy(sr, u_sh.at[didx], add=True)

    issue(0, 0)
    issue(1, 1)

    def body(t, carry):
        k0 = 2 * t
        wait(0)
        compute(k0, 0)
        issue(k0 + 2, 0)
        k1 = 2 * t + 1
        wait(1)
        compute(k1, 1)

        @pl.when(k1 + 2 < NCH)
        def _():
            issue(k1 + 2, 1)

        return carry

    lax.fori_loop(0, (NCH - 1) // 2, body, 0)
    wait(0)
    compute(NCH - 1, 0)
    plsc.subcore_barrier()

    pltpu.sync_copy(u_sh.at[pl.ds(sid * SLAB, SLAB)],
                    u_hbm.at[pl.ds(cid * PS + sid * SLAB, SLAB)])
    pltpu.sync_copy(s_loc, s_hbm.at[pl.ds(wid * PS, PS)])


def _finish_tc_body(u_ref, s_ref, w_ref, b_ref, o_ref):
    u = u_ref[0] + u_ref[1]
    s = jnp.sum(s_ref[...], axis=1)
    s = jnp.where(s == 0.0, 1.0, s)
    un = u / s[:, None]
    acc = lax.dot_general(un, w_ref[...],
                          dimension_numbers=(((1,), (1,)), ((), ())),
                          preferred_element_type=jnp.float32)
    o_ref[...] = jnp.maximum(acc + b_ref[...], 0.0)


RB = 1000


def _finish_tc(u_part, s_part, W, b):
    return pl.pallas_call(
        _finish_tc_body,
        grid=(N // RB,),
        in_specs=[
            pl.BlockSpec((2, RB, D), lambda i: (0, i, 0)),
            pl.BlockSpec((RB, NW), lambda i: (i, 0)),
            pl.BlockSpec((D, D), lambda i: (0, 0)),
            pl.BlockSpec((1, D), lambda i: (0, 0)),
        ],
        out_specs=pl.BlockSpec((RB, D), lambda i: (i, 0)),
        out_shape=jax.ShapeDtypeStruct((N, D), jnp.float32),
    )(u_part, s_part, W, b)


def kernel(feat_src, feat_dst, edge_index, W, b):
    src = edge_index[0].astype(jnp.int32)
    dst = edge_index[1].astype(jnp.int32)
    scores, tmax = _scores_sc(src, dst, feat_src, feat_dst)
    zu = jnp.zeros((PS, D), jnp.float32)
    zs = jnp.zeros((PS,), jnp.float32)
    u_part, s_part = _accum_sc(src, dst, feat_src, scores, tmax, zu, zs)
    u_part = u_part.reshape(NC, PS, D)[:, :N]
    s_part = s_part.reshape(NW, PS)[:, :N].T
    return _finish_tc(u_part, s_part, W, b.reshape(1, D))
